# baseline (speedup 1.0000x reference)
import functools

import jax
import jax.numpy as jnp
from jax import lax
from jax.experimental import pallas as pl
from jax.experimental.pallas import tpu as pltpu
from jax.experimental.pallas import tpu_sc as plsc

_NC, _NS = 2, 16
_WROWS = 24
_TROWS = 24
_NB = 3
_L = 16


def _ring(nrows, pos_in, x_in, x_out, compute, n_batches):
    pos_in().start()
    for k in range(_NB):
        x_in(k, k).start()
    pos_in().wait()

    def body(b, carry):
        slot = lax.rem(b, _NB)
        x_in(b, slot).wait()

        @pl.when(b >= _NB)
        def _():
            x_out(b - _NB, slot).wait()

        compute(slot, nrows)

        x_out(b, slot).start()

        @pl.when(b + _NB < n_batches)
        def _():
            x_in(b + _NB, slot).start()

        return carry

    lax.fori_loop(0, n_batches, body, jnp.int32(0))

    for b in range(n_batches - _NB, n_batches):
        x_out(b, b % _NB).wait()


def _sc_body(x_hbm, pos_hbm, o_hbm, posb, xb, ob, idxb, psem, xsem, osem):
    B, S, D = x_hbm.shape
    n_full = S // _WROWS
    tail_start = S - _TROWS
    wid = lax.axis_index("s") * _NC + lax.axis_index("c")

    def compute(slot, nrows):
        @plsc.parallel_loop(0, nrows, 1, unroll=2)
        def _(r):
            for j in range(D // _L):
                ob[slot, r, pl.ds(j * _L, _L)] = (
                    xb[slot, r, pl.ds(j * _L, _L)] + posb[r, pl.ds(j * _L, _L)])

    @pl.when(wid < n_full)
    def _():
        start = pl.multiple_of(wid * _WROWS, 8)

        def pos_in():
            return pltpu.make_async_copy(
                pos_hbm.at[pl.ds(start, _WROWS)],
                posb.at[pl.ds(0, _WROWS)], psem)

        def x_in(b, slot):
            return pltpu.make_async_copy(
                x_hbm.at[b, pl.ds(start, _WROWS)],
                xb.at[slot, pl.ds(0, _WROWS)], xsem.at[slot])

        def x_out(b, slot):
            return pltpu.make_async_copy(
                ob.at[slot, pl.ds(0, _WROWS)],
                o_hbm.at[b, pl.ds(start, _WROWS)], osem.at[slot])

        _ring(_WROWS, pos_in, x_in, x_out, compute, B)

    @pl.when(wid == n_full)
    def _():
        iota = lax.iota(jnp.int32, _L)
        idxb[pl.ds(0, _L)] = tail_start + iota
        idxb[pl.ds(_TROWS - _L, _L)] = tail_start + (_TROWS - _L) + iota

        def pos_in():
            return pltpu.make_async_copy(pos_hbm.at[idxb], posb, psem)

        def x_in(b, slot):
            return pltpu.make_async_copy(
                x_hbm.at[b].at[idxb], xb.at[slot], xsem.at[slot])

        def x_out(b, slot):
            return pltpu.make_async_copy(
                ob.at[slot], o_hbm.at[b].at[idxb], osem.at[slot])

        _ring(_TROWS, pos_in, x_in, x_out, compute, B)


def kernel(x, pos_table):
    B, S, D = x.shape
    run = functools.partial(
        pl.kernel,
        out_type=jax.ShapeDtypeStruct((B, S, D), x.dtype),
        mesh=plsc.VectorSubcoreMesh(core_axis_name="c", subcore_axis_name="s"),
        scratch_types=[
            pltpu.VMEM((_TROWS, D), x.dtype),
            pltpu.VMEM((_NB, _TROWS, D), x.dtype),
            pltpu.VMEM((_NB, _TROWS, D), x.dtype),
            pltpu.VMEM((_TROWS,), jnp.int32),
            pltpu.SemaphoreType.DMA,
            pltpu.SemaphoreType.DMA((_NB,)),
            pltpu.SemaphoreType.DMA((_NB,)),
        ],
    )(_sc_body)
    return run(x, pos_table)


# speedup vs baseline: 1.0324x; 1.0324x over previous
import functools

import jax
import jax.numpy as jnp
from jax import lax
from jax.experimental import pallas as pl
from jax.experimental.pallas import tpu as pltpu
from jax.experimental.pallas import tpu_sc as plsc

_NC, _NS = 2, 16
_WROWS = 24
_TROWS = 24
_NB = 3
_L = 16


def _ring(nrows, pos_in, x_in, x_out, compute, n_batches):
    pos_in().start()
    for k in range(_NB):
        x_in(k, k).start()
    pos_in().wait()

    def body(b, carry):
        slot = lax.rem(b, _NB)
        x_in(b, slot).wait()

        @pl.when(b >= _NB)
        def _():
            x_out(b - _NB, slot).wait()

        compute(slot, nrows)

        x_out(b, slot).start()

        @pl.when(b + _NB < n_batches)
        def _():
            x_in(b + _NB, slot).start()

        return carry

    lax.fori_loop(0, n_batches, body, jnp.int32(0))

    for b in range(n_batches - _NB, n_batches):
        x_out(b, b % _NB).wait()


_F = 28


def _sc_body(x_hbm, pos_hbm, o_hbm, posb, xb, ob, idxb, psem, xsem, osem):
    B = _F
    S, D = x_hbm.shape[1], x_hbm.shape[2]
    n_full = S // _WROWS
    tail_start = S - _TROWS
    wid = lax.axis_index("s") * _NC + lax.axis_index("c")

    def compute(slot, nrows):
        @plsc.parallel_loop(0, nrows, 1, unroll=2)
        def _(r):
            for j in range(D // _L):
                ob[slot, r, pl.ds(j * _L, _L)] = (
                    xb[slot, r, pl.ds(j * _L, _L)] + posb[r, pl.ds(j * _L, _L)])

    @pl.when(wid < n_full)
    def _():
        start = pl.multiple_of(wid * _WROWS, 8)

        def pos_in():
            return pltpu.make_async_copy(
                pos_hbm.at[pl.ds(start, _WROWS)],
                posb.at[pl.ds(0, _WROWS)], psem)

        def x_in(b, slot):
            return pltpu.make_async_copy(
                x_hbm.at[b, pl.ds(start, _WROWS)],
                xb.at[slot, pl.ds(0, _WROWS)], xsem.at[slot])

        def x_out(b, slot):
            return pltpu.make_async_copy(
                ob.at[slot, pl.ds(0, _WROWS)],
                o_hbm.at[b, pl.ds(start, _WROWS)], osem.at[slot])

        _ring(_WROWS, pos_in, x_in, x_out, compute, B)

    @pl.when(wid == n_full)
    def _():
        iota = lax.iota(jnp.int32, _L)
        idxb[pl.ds(0, _L)] = tail_start + iota
        idxb[pl.ds(_TROWS - _L, _L)] = tail_start + (_TROWS - _L) + iota

        def pos_in():
            return pltpu.make_async_copy(pos_hbm.at[idxb], posb, psem)

        def x_in(b, slot):
            return pltpu.make_async_copy(
                x_hbm.at[b].at[idxb], xb.at[slot], xsem.at[slot])

        def x_out(b, slot):
            return pltpu.make_async_copy(
                ob.at[slot], o_hbm.at[b].at[idxb], osem.at[slot])

        _ring(_TROWS, pos_in, x_in, x_out, compute, B)


def _tc_body(x_ref, pos_ref, o_ref):
    s = o_ref.shape[1]
    o_ref[...] = x_ref[...] + pos_ref[:s][None, :, :]


def kernel(x, pos_table):
    B, S, D = x.shape
    run = functools.partial(
        pl.kernel,
        out_type=jax.ShapeDtypeStruct((_F, S, D), x.dtype),
        mesh=plsc.VectorSubcoreMesh(core_axis_name="c", subcore_axis_name="s"),
        scratch_types=[
            pltpu.VMEM((_TROWS, D), x.dtype),
            pltpu.VMEM((_NB, _TROWS, D), x.dtype),
            pltpu.VMEM((_NB, _TROWS, D), x.dtype),
            pltpu.VMEM((_TROWS,), jnp.int32),
            pltpu.SemaphoreType.DMA,
            pltpu.SemaphoreType.DMA((_NB,)),
            pltpu.SemaphoreType.DMA((_NB,)),
        ],
    )(_sc_body)
    sc_out = run(x, pos_table)

    bblk = 4
    tc_out = pl.pallas_call(
        _tc_body,
        grid=((B - _F) // bblk,),
        in_specs=[
            pl.BlockSpec((bblk, S, D), lambda i: (_F // bblk + i, 0, 0)),
            pl.BlockSpec(pos_table.shape, lambda i: (0, 0)),
        ],
        out_specs=pl.BlockSpec((bblk, S, D), lambda i: (_F // bblk + i, 0, 0)),
        out_shape=jax.ShapeDtypeStruct((B, S, D), x.dtype),
        compiler_params=pltpu.CompilerParams(
            dimension_semantics=("arbitrary",),
        ),
    )(x, pos_table)
    return lax.dynamic_update_slice(tc_out, sc_out, (0, 0, 0))


# speedup vs baseline: 1.0644x; 1.0309x over previous
import functools

import jax
import jax.numpy as jnp
from jax import lax
from jax.experimental import pallas as pl
from jax.experimental.pallas import tpu as pltpu
from jax.experimental.pallas import tpu_sc as plsc

_NC, _NS = 2, 16
_WROWS = 24
_TROWS = 24
_NB = 3
_L = 16


def _ring(nrows, pos_in, x_in, x_out, compute, n_batches):
    pos_in().start()
    for k in range(_NB):
        x_in(k, k).start()
    pos_in().wait()

    def body(b, carry):
        slot = lax.rem(b, _NB)
        x_in(b, slot).wait()

        @pl.when(b >= _NB)
        def _():
            x_out(b - _NB, slot).wait()

        compute(slot, nrows)

        x_out(b, slot).start()

        @pl.when(b + _NB < n_batches)
        def _():
            x_in(b + _NB, slot).start()

        return carry

    lax.fori_loop(0, n_batches, body, jnp.int32(0))

    for b in range(n_batches - _NB, n_batches):
        x_out(b, b % _NB).wait()


_F = 20


def _sc_body(x_hbm, pos_hbm, o_hbm, posb, xb, ob, idxb, psem, xsem, osem):
    B = _F
    S, D = x_hbm.shape[1], x_hbm.shape[2]
    n_full = S // _WROWS
    tail_start = S - _TROWS
    wid = lax.axis_index("s") * _NC + lax.axis_index("c")

    def compute(slot, nrows):
        @plsc.parallel_loop(0, nrows, 1, unroll=2)
        def _(r):
            for j in range(D // _L):
                ob[slot, r, pl.ds(j * _L, _L)] = (
                    xb[slot, r, pl.ds(j * _L, _L)] + posb[r, pl.ds(j * _L, _L)])

    @pl.when(wid < n_full)
    def _():
        start = pl.multiple_of(wid * _WROWS, 8)

        def pos_in():
            return pltpu.make_async_copy(
                pos_hbm.at[pl.ds(start, _WROWS)],
                posb.at[pl.ds(0, _WROWS)], psem)

        def x_in(b, slot):
            return pltpu.make_async_copy(
                x_hbm.at[b, pl.ds(start, _WROWS)],
                xb.at[slot, pl.ds(0, _WROWS)], xsem.at[slot])

        def x_out(b, slot):
            return pltpu.make_async_copy(
                ob.at[slot, pl.ds(0, _WROWS)],
                o_hbm.at[b, pl.ds(start, _WROWS)], osem.at[slot])

        _ring(_WROWS, pos_in, x_in, x_out, compute, B)

    @pl.when(wid == n_full)
    def _():
        iota = lax.iota(jnp.int32, _L)
        idxb[pl.ds(0, _L)] = tail_start + iota
        idxb[pl.ds(_TROWS - _L, _L)] = tail_start + (_TROWS - _L) + iota

        def pos_in():
            return pltpu.make_async_copy(pos_hbm.at[idxb], posb, psem)

        def x_in(b, slot):
            return pltpu.make_async_copy(
                x_hbm.at[b].at[idxb], xb.at[slot], xsem.at[slot])

        def x_out(b, slot):
            return pltpu.make_async_copy(
                ob.at[slot], o_hbm.at[b].at[idxb], osem.at[slot])

        _ring(_TROWS, pos_in, x_in, x_out, compute, B)


def _tc_body(x_ref, pos_ref, o_ref):
    s = o_ref.shape[1]
    o_ref[...] = x_ref[...] + pos_ref[:s][None, :, :]


def kernel(x, pos_table):
    B, S, D = x.shape
    run = functools.partial(
        pl.kernel,
        out_type=jax.ShapeDtypeStruct((_F, S, D), x.dtype),
        mesh=plsc.VectorSubcoreMesh(core_axis_name="c", subcore_axis_name="s"),
        scratch_types=[
            pltpu.VMEM((_TROWS, D), x.dtype),
            pltpu.VMEM((_NB, _TROWS, D), x.dtype),
            pltpu.VMEM((_NB, _TROWS, D), x.dtype),
            pltpu.VMEM((_TROWS,), jnp.int32),
            pltpu.SemaphoreType.DMA,
            pltpu.SemaphoreType.DMA((_NB,)),
            pltpu.SemaphoreType.DMA((_NB,)),
        ],
    )(_sc_body)
    sc_out = run(x, pos_table)

    bblk = 4
    tc_out = pl.pallas_call(
        _tc_body,
        grid=((B - _F) // bblk,),
        in_specs=[
            pl.BlockSpec((bblk, S, D), lambda i: (_F // bblk + i, 0, 0)),
            pl.BlockSpec(pos_table.shape, lambda i: (0, 0)),
        ],
        out_specs=pl.BlockSpec((bblk, S, D), lambda i: (_F // bblk + i, 0, 0)),
        out_shape=jax.ShapeDtypeStruct((B, S, D), x.dtype),
        compiler_params=pltpu.CompilerParams(
            dimension_semantics=("arbitrary",),
        ),
    )(x, pos_table)
    return lax.dynamic_update_slice(tc_out, sc_out, (0, 0, 0))


# speedup vs baseline: 1.0845x; 1.0189x over previous
import functools

import jax
import jax.numpy as jnp
from jax import lax
from jax.experimental import pallas as pl
from jax.experimental.pallas import tpu as pltpu
from jax.experimental.pallas import tpu_sc as plsc

_NC, _NS = 2, 16
_WROWS = 24
_TROWS = 24
_NB = 3
_L = 16


def _ring(nrows, pos_in, x_in, x_out, compute, n_batches):
    pos_in().start()
    for k in range(_NB):
        x_in(k, k).start()
    pos_in().wait()

    def body(b, carry):
        slot = lax.rem(b, _NB)
        x_in(b, slot).wait()

        @pl.when(b >= _NB)
        def _():
            x_out(b - _NB, slot).wait()

        compute(slot, nrows)

        x_out(b, slot).start()

        @pl.when(b + _NB < n_batches)
        def _():
            x_in(b + _NB, slot).start()

        return carry

    lax.fori_loop(0, n_batches, body, jnp.int32(0))

    for b in range(n_batches - _NB, n_batches):
        x_out(b, b % _NB).wait()


_F = 16


def _sc_body(x_hbm, pos_hbm, o_hbm, posb, xb, ob, idxb, psem, xsem, osem):
    B = _F
    S, D = x_hbm.shape[1], x_hbm.shape[2]
    n_full = S // _WROWS
    tail_start = S - _TROWS
    wid = lax.axis_index("s") * _NC + lax.axis_index("c")

    def compute(slot, nrows):
        @plsc.parallel_loop(0, nrows, 1, unroll=2)
        def _(r):
            for j in range(D // _L):
                ob[slot, r, pl.ds(j * _L, _L)] = (
                    xb[slot, r, pl.ds(j * _L, _L)] + posb[r, pl.ds(j * _L, _L)])

    @pl.when(wid < n_full)
    def _():
        start = pl.multiple_of(wid * _WROWS, 8)

        def pos_in():
            return pltpu.make_async_copy(
                pos_hbm.at[pl.ds(start, _WROWS)],
                posb.at[pl.ds(0, _WROWS)], psem)

        def x_in(b, slot):
            return pltpu.make_async_copy(
                x_hbm.at[b, pl.ds(start, _WROWS)],
                xb.at[slot, pl.ds(0, _WROWS)], xsem.at[slot])

        def x_out(b, slot):
            return pltpu.make_async_copy(
                ob.at[slot, pl.ds(0, _WROWS)],
                o_hbm.at[b, pl.ds(start, _WROWS)], osem.at[slot])

        _ring(_WROWS, pos_in, x_in, x_out, compute, B)

    @pl.when(wid == n_full)
    def _():
        iota = lax.iota(jnp.int32, _L)
        idxb[pl.ds(0, _L)] = tail_start + iota
        idxb[pl.ds(_TROWS - _L, _L)] = tail_start + (_TROWS - _L) + iota

        def pos_in():
            return pltpu.make_async_copy(pos_hbm.at[idxb], posb, psem)

        def x_in(b, slot):
            return pltpu.make_async_copy(
                x_hbm.at[b].at[idxb], xb.at[slot], xsem.at[slot])

        def x_out(b, slot):
            return pltpu.make_async_copy(
                ob.at[slot], o_hbm.at[b].at[idxb], osem.at[slot])

        _ring(_TROWS, pos_in, x_in, x_out, compute, B)


def _tc_body(x_ref, pos_ref, o_ref):
    s = o_ref.shape[1]
    o_ref[...] = x_ref[...] + pos_ref[:s][None, :, :]


def kernel(x, pos_table):
    B, S, D = x.shape
    run = functools.partial(
        pl.kernel,
        out_type=jax.ShapeDtypeStruct((_F, S, D), x.dtype),
        mesh=plsc.VectorSubcoreMesh(core_axis_name="c", subcore_axis_name="s"),
        scratch_types=[
            pltpu.VMEM((_TROWS, D), x.dtype),
            pltpu.VMEM((_NB, _TROWS, D), x.dtype),
            pltpu.VMEM((_NB, _TROWS, D), x.dtype),
            pltpu.VMEM((_TROWS,), jnp.int32),
            pltpu.SemaphoreType.DMA,
            pltpu.SemaphoreType.DMA((_NB,)),
            pltpu.SemaphoreType.DMA((_NB,)),
        ],
    )(_sc_body)
    sc_out = run(x, pos_table)

    bblk = 4
    tc_out = pl.pallas_call(
        _tc_body,
        grid=((B - _F) // bblk,),
        in_specs=[
            pl.BlockSpec((bblk, S, D), lambda i: (_F // bblk + i, 0, 0)),
            pl.BlockSpec(pos_table.shape, lambda i: (0, 0)),
        ],
        out_specs=pl.BlockSpec((bblk, S, D), lambda i: (_F // bblk + i, 0, 0)),
        out_shape=jax.ShapeDtypeStruct((B, S, D), x.dtype),
        compiler_params=pltpu.CompilerParams(
            dimension_semantics=("arbitrary",),
        ),
    )(x, pos_table)
    return lax.dynamic_update_slice(tc_out, sc_out, (0, 0, 0))
